# baseline (speedup 1.0000x reference)
import functools

import jax
import jax.numpy as jnp
from jax import lax
from jax.experimental import pallas as pl
from jax.experimental.pallas import tpu as pltpu
from jax.experimental.pallas import tpu_sc as plsc

NUM_WORKERS = 32
LT = 8
BC = 512


def _build_sc_gather(B, L, D, VPAD):
    n_bc = B // BC
    n_units = (L // LT) * n_bc
    units_per_w = n_units // NUM_WORKERS
    groups = BC // 16
    mesh = plsc.VectorSubcoreMesh(core_axis_name="c", subcore_axis_name="s")

    @functools.partial(
        pl.kernel,
        mesh=mesh,
        out_type=jax.ShapeDtypeStruct((D, L, B), jnp.float32),
        compiler_params=pltpu.CompilerParams(needs_layout_passes=False),
        scratch_types=[
            [pltpu.VMEM((VPAD,), jnp.float32) for _ in range(D)],
            pltpu.VMEM((LT, BC), jnp.int32),
            pltpu.VMEM((LT, BC), jnp.int32),
            pltpu.VMEM((D, LT, BC), jnp.float32),
            pltpu.VMEM((D, LT, BC), jnp.float32),
            pltpu.SemaphoreType.DMA,
            pltpu.SemaphoreType.DMA,
            pltpu.SemaphoreType.DMA,
            pltpu.SemaphoreType.DMA,
        ],
    )
    def gather_kernel(
        table_hbm, idx_hbm, out_hbm,
        tabs, idx_v0, idx_v1, slab_v0, slab_v1,
        sem_in0, sem_in1, sem_out0, sem_out1,
    ):
        wid = lax.axis_index("s") * 2 + lax.axis_index("c")
        for j in range(D):
            pltpu.sync_copy(table_hbm.at[pl.ds(j * VPAD, VPAD)], tabs[j])
        idx_bufs = (idx_v0, idx_v1)
        slab_bufs = (slab_v0, slab_v1)
        in_sems = (sem_in0, sem_in1)
        out_sems = (sem_out0, sem_out1)

        def idx_src(u):
            unit = wid * units_per_w + u
            lt = unit // n_bc
            bc = unit % n_bc
            return idx_hbm.at[pl.ds(lt * LT, LT), pl.ds(bc * BC, BC)]

        def out_dst(u):
            unit = wid * units_per_w + u
            lt = unit // n_bc
            bc = unit % n_bc
            return out_hbm.at[:, pl.ds(lt * LT, LT), pl.ds(bc * BC, BC)]

        def compute(idx_v, slab_v):
            @plsc.parallel_loop(0, LT * groups, unroll=8)
            def group_body(g):
                r = g // groups
                cc = (g % groups) * 16
                rows = idx_v[r, pl.ds(cc, 16)]
                for j in range(D):
                    vals = plsc.load_gather(tabs[j], [rows])
                    slab_v[j, r, pl.ds(cc, 16)] = vals

        def step(u, b):
            idx_v, slab_v = idx_bufs[b], slab_bufs[b]
            sem_in, sem_out = in_sems[b], out_sems[b]
            pltpu.make_async_copy(idx_src(u), idx_v, sem_in).wait()

            @pl.when(u >= 2)
            def _():
                pltpu.make_async_copy(slab_v, out_dst(u - 2), sem_out).wait()

            compute(idx_v, slab_v)
            pltpu.async_copy(slab_v, out_dst(u), sem_out)

            @pl.when(u + 2 < units_per_w)
            def _():
                pltpu.async_copy(idx_src(u + 2), idx_v, sem_in)

        pltpu.async_copy(idx_src(0), idx_v0, sem_in0)
        pltpu.async_copy(idx_src(1), idx_v1, sem_in1)

        def outer_body(o, carry):
            step(o * 2, 0)
            step(o * 2 + 1, 1)
            return carry

        lax.fori_loop(0, units_per_w // 2, outer_body, 0)
        if units_per_w % 2:
            step(units_per_w - 1, 0)
        pltpu.make_async_copy(
            slab_bufs[units_per_w % 2], out_dst(units_per_w - 2),
            out_sems[units_per_w % 2],
        ).wait()
        pltpu.make_async_copy(
            slab_bufs[1 - units_per_w % 2], out_dst(units_per_w - 1),
            out_sems[1 - units_per_w % 2],
        ).wait()

    return gather_kernel


def kernel(atomic_numbers, mask_weight):
    B, L = atomic_numbers.shape
    V, D = mask_weight.shape
    VPAD = 128
    idx_t = atomic_numbers.T
    cols = jnp.zeros((D, VPAD), jnp.float32).at[:, :V].set(mask_weight.T)
    out_dlb = _build_sc_gather(B, L, D, VPAD)(cols.reshape(D * VPAD), idx_t)
    return out_dlb.transpose(2, 1, 0)


# speedup vs baseline: 1.0753x; 1.0753x over previous
import functools

import jax
import jax.numpy as jnp
from jax import lax
from jax.experimental import pallas as pl
from jax.experimental.pallas import tpu as pltpu
from jax.experimental.pallas import tpu_sc as plsc

NUM_WORKERS = 32
LT = 8
BC = 512


def _build_sc_gather(B, L, D, VPAD):
    n_bc = B // BC
    n_units = (L // LT) * n_bc
    units_per_w = n_units // NUM_WORKERS
    groups = BC // 16
    mesh = plsc.VectorSubcoreMesh(core_axis_name="c", subcore_axis_name="s")

    @functools.partial(
        pl.kernel,
        mesh=mesh,
        out_type=jax.ShapeDtypeStruct((D, L, B), jnp.float32),
        compiler_params=pltpu.CompilerParams(needs_layout_passes=False),
        scratch_types=[
            [pltpu.VMEM((VPAD,), jnp.float32) for _ in range(D)],
            pltpu.VMEM((LT, BC), jnp.int32),
            pltpu.VMEM((LT, BC), jnp.int32),
            pltpu.VMEM((D, LT, BC), jnp.float32),
            pltpu.VMEM((D, LT, BC), jnp.float32),
            pltpu.SemaphoreType.DMA,
            pltpu.SemaphoreType.DMA,
            pltpu.SemaphoreType.DMA,
            pltpu.SemaphoreType.DMA,
            pltpu.SemaphoreType.DMA,
        ],
    )
    def gather_kernel(
        table_hbm, idx_hbm, out_hbm,
        tabs, idx_v0, idx_v1, slab_v0, slab_v1,
        sem_in0, sem_in1, sem_out0, sem_out1, sem_tab,
    ):
        wid = lax.axis_index("s") * 2 + lax.axis_index("c")
        for j in range(D):
            pltpu.async_copy(table_hbm.at[pl.ds(j * VPAD, VPAD)], tabs[j], sem_tab)
        idx_bufs = (idx_v0, idx_v1)
        slab_bufs = (slab_v0, slab_v1)
        in_sems = (sem_in0, sem_in1)
        out_sems = (sem_out0, sem_out1)

        def idx_src(u):
            unit = wid * units_per_w + u
            lt = unit // n_bc
            bc = unit % n_bc
            return idx_hbm.at[pl.ds(lt * LT, LT), pl.ds(bc * BC, BC)]

        def out_dst(u):
            unit = wid * units_per_w + u
            lt = unit // n_bc
            bc = unit % n_bc
            return out_hbm.at[:, pl.ds(lt * LT, LT), pl.ds(bc * BC, BC)]

        def compute(idx_v, slab_v):
            @plsc.parallel_loop(0, LT * groups, unroll=4)
            def group_body(g):
                r = g // groups
                cc = (g % groups) * 16
                rows = idx_v[r, pl.ds(cc, 16)]
                for j in range(D):
                    vals = plsc.load_gather(tabs[j], [rows])
                    slab_v[j, r, pl.ds(cc, 16)] = vals

        def step(u, b):
            idx_v, slab_v = idx_bufs[b], slab_bufs[b]
            sem_in, sem_out = in_sems[b], out_sems[b]
            pltpu.make_async_copy(idx_src(u), idx_v, sem_in).wait()

            @pl.when(u >= 2)
            def _():
                pltpu.make_async_copy(slab_v, out_dst(u - 2), sem_out).wait()

            compute(idx_v, slab_v)
            pltpu.async_copy(slab_v, out_dst(u), sem_out)

            @pl.when(u + 2 < units_per_w)
            def _():
                pltpu.async_copy(idx_src(u + 2), idx_v, sem_in)

        pltpu.async_copy(idx_src(0), idx_v0, sem_in0)
        pltpu.async_copy(idx_src(1), idx_v1, sem_in1)
        for j in range(D):
            pltpu.make_async_copy(
                table_hbm.at[pl.ds(j * VPAD, VPAD)], tabs[j], sem_tab
            ).wait()

        def outer_body(o, carry):
            step(o * 2, 0)
            step(o * 2 + 1, 1)
            return carry

        lax.fori_loop(0, units_per_w // 2, outer_body, 0)
        if units_per_w % 2:
            step(units_per_w - 1, 0)
        pltpu.make_async_copy(
            slab_bufs[units_per_w % 2], out_dst(units_per_w - 2),
            out_sems[units_per_w % 2],
        ).wait()
        pltpu.make_async_copy(
            slab_bufs[1 - units_per_w % 2], out_dst(units_per_w - 1),
            out_sems[1 - units_per_w % 2],
        ).wait()

    return gather_kernel


def kernel(atomic_numbers, mask_weight):
    B, L = atomic_numbers.shape
    V, D = mask_weight.shape
    VPAD = 128
    idx_t = atomic_numbers.T
    cols = jnp.zeros((D, VPAD), jnp.float32).at[:, :V].set(mask_weight.T)
    out_dlb = _build_sc_gather(B, L, D, VPAD)(cols.reshape(D * VPAD), idx_t)
    return out_dlb.transpose(2, 1, 0)
